# baseline (speedup 1.0000x reference)
import functools

import jax
import jax.numpy as jnp
from jax import lax
from jax.experimental import pallas as pl
from jax.experimental.pallas import tpu as pltpu
from jax.experimental.pallas import tpu_sc as plsc

NC, NS, L = 2, 16, 16
NW = NC * NS
B, S, D = 4, 4096, 1024
SCALE = 32.0
PW = S // NW
CP = 16
NCHUNK = PW // CP
NJ = NCHUNK * B
NB = 5
AH = 3
GROUPS = D // L

_mesh = plsc.VectorSubcoreMesh(core_axis_name="c", subcore_axis_name="s")


@functools.partial(
    pl.kernel,
    out_type=jax.ShapeDtypeStruct((B, S, D), jnp.float32),
    mesh=_mesh,
    scratch_types=[
        pltpu.VMEM((B, NCHUNK, CP), jnp.int32),
        pltpu.VMEM((2, CP, D), jnp.float32),
        pltpu.VMEM((NB, CP, D), jnp.float32),
        pltpu.SemaphoreType.DMA,
        pltpu.SemaphoreType.DMA((2,)),
        pltpu.SemaphoreType.DMA((NB,)),
        pltpu.SemaphoreType.DMA((NB,)),
    ],
)
def _emb_kernel(inputs_hbm, tok_hbm, pos_hbm, out_hbm, idx_v, posv, tokv,
                si, spv, sgv, sov):
    tok = [tokv.at[k] for k in range(NB)]
    sp = [spv.at[k] for k in range(2)]
    sg = [sgv.at[k] for k in range(NB)]
    so = [sov.at[k] for k in range(NB)]
    posb = [posv.at[0], posv.at[1]]

    wid = lax.axis_index("s") * NC + lax.axis_index("c")
    p0 = wid * PW

    hidx = []
    for b in range(B):
        h = pltpu.make_async_copy(
            inputs_hbm.at[b, pl.ds(pl.multiple_of(p0 // CP, NCHUNK), NCHUNK)],
            idx_v.at[b], si)
        h.start()
        hidx.append(h)
    idx_ready = [False] * B

    def start_pos(c):
        h = pltpu.make_async_copy(
            pos_hbm.at[pl.ds(p0 + c * CP, CP)], posb[c % 2], sp[c % 2])
        h.start()
        return h

    def start_gather(j):
        c, b = j // B, j % B
        nb = j % NB
        if not idx_ready[b]:
            hidx[b].wait()
            idx_ready[b] = True
        h = pltpu.make_async_copy(
            tok_hbm.at[idx_v.at[b, c]], tok[nb], sg[nb])
        h.start()
        return h

    def start_out(j):
        c, b = j // B, j % B
        nb = j % NB
        h = pltpu.make_async_copy(
            tok[nb], out_hbm.at[b, pl.ds(p0 + c * CP, CP)], so[nb])
        h.start()
        return h

    hp = [start_pos(0), start_pos(1)]
    hg = [None] * NB
    ho = [None] * NB
    for k in range(AH):
        hg[k % NB] = start_gather(k)

    for jp in range(0, NJ, 2):
        c, b0 = jp // B, jp % B
        n0, n1 = jp % NB, (jp + 1) % NB
        for jn in (jp + AH, jp + AH + 1):
            if jn < NJ:
                cn, bn = jn // B, jn % B
                tb = jn % NB
                if ho[tb] is not None:
                    ho[tb].wait()
                if bn == 0 and cn >= 2:
                    hp[cn % 2] = start_pos(cn)
                hg[tb] = start_gather(jn)

        hg[n0].wait()
        hg[n1].wait()
        if b0 == 0:
            hp[c % 2].wait()

        t0, t1 = tok[n0], tok[n1]
        pbuf = posb[c % 2]

        @pl.loop(0, CP * GROUPS, unroll=4)
        def _fma(g):
            r = g // GROUPS
            sl = pl.ds((g % GROUPS) * L, L)
            p = pbuf[r, sl]
            t0[r, sl] = t0[r, sl] * SCALE + p
            t1[r, sl] = t1[r, sl] * SCALE + p

        ho[n0] = start_out(jp)
        ho[n1] = start_out(jp + 1)

    for nb in range(NB):
        if ho[nb] is not None:
            ho[nb].wait()


def kernel(inputs, tok_table, pos_table):
    idx = inputs.astype(jnp.int32).reshape(B, S // CP, CP)
    return _emb_kernel(idx, tok_table, pos_table)


# speedup vs baseline: 1.0066x; 1.0066x over previous
import functools

import jax
import jax.numpy as jnp
from jax import lax
from jax.experimental import pallas as pl
from jax.experimental.pallas import tpu as pltpu
from jax.experimental.pallas import tpu_sc as plsc

NC, NS, L = 2, 16, 16
NW = NC * NS
B, S, D = 4, 4096, 1024
SCALE = 32.0
PW = S // NW
CP = 16
NCHUNK = PW // CP
NJ = NCHUNK * B
NB = 5
AH = 3
GROUPS = D // L

_mesh = plsc.VectorSubcoreMesh(core_axis_name="c", subcore_axis_name="s")


@functools.partial(
    pl.kernel,
    out_type=jax.ShapeDtypeStruct((B, S, D), jnp.float32),
    mesh=_mesh,
    scratch_types=[
        pltpu.VMEM((B, NCHUNK, CP), jnp.int32),
        pltpu.VMEM((2, CP, D), jnp.float32),
        pltpu.VMEM((NB, CP, D), jnp.float32),
        pltpu.SemaphoreType.DMA,
        pltpu.SemaphoreType.DMA((2,)),
        pltpu.SemaphoreType.DMA((NB,)),
        pltpu.SemaphoreType.DMA((NB,)),
    ],
)
def _emb_kernel(inputs_hbm, tok_hbm, pos_hbm, out_hbm, idx_v, posv, tokv,
                si, spv, sgv, sov):
    tok = [tokv.at[k] for k in range(NB)]
    sp = [spv.at[k] for k in range(2)]
    sg = [sgv.at[k] for k in range(NB)]
    so = [sov.at[k] for k in range(NB)]
    posb = [posv.at[0], posv.at[1]]

    wid = lax.axis_index("s") * NC + lax.axis_index("c")
    p0 = wid * PW

    hidx = []
    for b in range(B):
        h = pltpu.make_async_copy(
            inputs_hbm.at[b, pl.ds(pl.multiple_of(p0 // CP, NCHUNK), NCHUNK)],
            idx_v.at[b], si)
        h.start()
        hidx.append(h)
    idx_ready = [False] * B

    def start_pos(c):
        h = pltpu.make_async_copy(
            pos_hbm.at[pl.ds(p0 + c * CP, CP)], posb[c % 2], sp[c % 2])
        h.start()
        return h

    def start_gather(j):
        c, b = j // B, j % B
        nb = j % NB
        if not idx_ready[b]:
            hidx[b].wait()
            idx_ready[b] = True
        h = pltpu.make_async_copy(
            tok_hbm.at[idx_v.at[b, c]], tok[nb], sg[nb])
        h.start()
        return h

    def start_out(j):
        c, b = j // B, j % B
        nb = j % NB
        h = pltpu.make_async_copy(
            tok[nb], out_hbm.at[b, pl.ds(p0 + c * CP, CP)], so[nb])
        h.start()
        return h

    hp = [start_pos(0), start_pos(1)]
    hg = [None] * NB
    ho = [None] * NB
    for k in range(AH):
        hg[k % NB] = start_gather(k)

    for j in range(NJ):
        c, b = j // B, j % B
        nb = j % NB
        jn = j + AH
        if jn < NJ:
            cn, bn = jn // B, jn % B
            tb = jn % NB
            if ho[tb] is not None:
                ho[tb].wait()
            if bn == 0 and cn >= 2:
                hp[cn % 2] = start_pos(cn)
            hg[tb] = start_gather(jn)

        hg[nb].wait()
        if b == 0:
            hp[c % 2].wait()

        tbuf = tok[nb]
        pbuf = posb[c % 2]

        @pl.loop(0, CP * GROUPS, unroll=8)
        def _fma(g):
            r = g // GROUPS
            off = (g % GROUPS) * L
            t = tbuf[r, pl.ds(off, L)]
            p = pbuf[r, pl.ds(off, L)]
            tbuf[r, pl.ds(off, L)] = t * SCALE + p

        ho[nb] = start_out(j)

    for nb in range(NB):
        if ho[nb] is not None:
            ho[nb].wait()


def kernel(inputs, tok_table, pos_table):
    idx = inputs.astype(jnp.int32).reshape(B, S // CP, CP)
    return _emb_kernel(idx, tok_table, pos_table)
